# Initial kernel scaffold; baseline (speedup 1.0000x reference)
import jax
import jax.numpy as jnp
from jax.experimental import pallas as pl


def kernel(x, edge_attr, edge_index, W_node, b_node, W_edge, b_edge, Wa1, ba1, Wa2, ba2):
    raise NotImplementedError("write your pallas kernel here")


# speedup vs baseline: 1.7851x; 1.7851x over previous
import functools

import jax
import jax.numpy as jnp
from jax import lax
from jax.experimental import pallas as pl
from jax.experimental.pallas import tpu as pltpu
from jax.experimental.pallas import tpu_sc as plsc

NC = 2
NS = 16
LANES = 16


def _tc_body(x_ref, wn_ref, bn_ref, wt_ref, wb_ref, ba1_ref, a_ref, b_ref):
    h = jnp.dot(x_ref[...], wn_ref[...], preferred_element_type=jnp.float32)
    h = h + bn_ref[...]
    a_ref[...] = jnp.dot(h, wt_ref[...], preferred_element_type=jnp.float32)
    b_ref[...] = (
        jnp.dot(h, wb_ref[...], preferred_element_type=jnp.float32) + ba1_ref[...]
    )


def _tc_stage(xp, wn, bn, wt, wb, ba1):
    n, dpad = xp.shape
    ch = wt.shape[1]
    blk = 1000
    grid = n // blk
    return pl.pallas_call(
        _tc_body,
        grid=(grid,),
        in_specs=[
            pl.BlockSpec((blk, dpad), lambda i: (i, 0)),
            pl.BlockSpec((dpad, ch), lambda i: (0, 0)),
            pl.BlockSpec((1, ch), lambda i: (0, 0)),
            pl.BlockSpec((ch, ch), lambda i: (0, 0)),
            pl.BlockSpec((ch, ch), lambda i: (0, 0)),
            pl.BlockSpec((1, ch), lambda i: (0, 0)),
        ],
        out_specs=[
            pl.BlockSpec((blk, ch), lambda i: (i, 0)),
            pl.BlockSpec((blk, ch), lambda i: (i, 0)),
        ],
        out_shape=[
            jax.ShapeDtypeStruct((n, ch), jnp.float32),
            jax.ShapeDtypeStruct((n, ch), jnp.float32),
        ],
    )(xp, wn, bn, wt, wb, ba1)


def _make_sc_stage(n_nodes, ch, e_edges):
    nw = NC * NS
    ew = e_edges // nw
    c_chunk = 80
    nchunk = ew // c_chunk
    assert ew * nw == e_edges and nchunk * c_chunk == ew

    mesh = plsc.VectorSubcoreMesh(
        core_axis_name="c", subcore_axis_name="s", num_cores=NC, num_subcores=NS
    )

    @functools.partial(
        pl.kernel,
        out_type=jax.ShapeDtypeStruct((e_edges,), jnp.float32),
        mesh=mesh,
        compiler_params=pltpu.CompilerParams(needs_layout_passes=False),
        scratch_types=[
            pltpu.VMEM((c_chunk,), jnp.int32),
            pltpu.VMEM((c_chunk,), jnp.int32),
            pltpu.VMEM((c_chunk, ch), jnp.float32),
            pltpu.VMEM((c_chunk, ch), jnp.float32),
            pltpu.VMEM((ch + LANES,), jnp.float32),
            pltpu.VMEM((c_chunk,), jnp.float32),
            pltpu.SemaphoreType.DMA,
            pltpu.SemaphoreType.DMA,
        ],
    )
    def sc_stage(a_hbm, b_hbm, src_hbm, dst_hbm, par_hbm, out_hbm,
                 src_v, dst_v, arow_v, brow_v, par_v, out_v, sem_a, sem_b):
        cid = lax.axis_index("c")
        sid = lax.axis_index("s")
        wid = sid * NC + cid
        base = wid * ew
        ngrp = c_chunk // LANES
        pltpu.sync_copy(par_hbm, par_v)
        ba2v = plsc.load_gather(par_v, [jnp.full((LANES,), ch, jnp.int32)])
        rows = [g * LANES + lax.iota(jnp.int32, LANES) for g in range(ngrp)]
        zero = jnp.zeros((LANES,), jnp.float32)

        def chunk_body(i, carry):
            off = base + i * c_chunk
            pltpu.sync_copy(src_hbm.at[pl.ds(off, c_chunk)], src_v)
            pltpu.sync_copy(dst_hbm.at[pl.ds(off, c_chunk)], dst_v)
            cp_a = pltpu.async_copy(a_hbm.at[src_v], arow_v, sem_a)
            cp_b = pltpu.async_copy(b_hbm.at[dst_v], brow_v, sem_b)
            cp_a.wait()
            cp_b.wait()

            def ch_body(c, accs):
                colc = jnp.full((LANES,), c, jnp.int32)
                wv = plsc.load_gather(par_v, [colc])
                out = []
                for g in range(ngrp):
                    av = plsc.load_gather(arow_v, [rows[g], colc])
                    bv = plsc.load_gather(brow_v, [rows[g], colc])
                    z = av + bv
                    z = jnp.maximum(z, z * jnp.float32(0.01))
                    out.append(accs[g] + z * wv)
                return tuple(out)

            accs = lax.fori_loop(0, ch, ch_body, (zero,) * ngrp, unroll=4)
            for g in range(ngrp):
                out_v[pl.ds(g * LANES, LANES)] = accs[g] + ba2v
            pltpu.sync_copy(out_v, out_hbm.at[pl.ds(off, c_chunk)])
            return carry

        lax.fori_loop(0, nchunk, chunk_body, 0)

    return sc_stage


def kernel(x, edge_attr, edge_index, W_node, b_node, W_edge, b_edge,
           Wa1, ba1, Wa2, ba2):
    n, d_node = x.shape
    ch = W_node.shape[1]
    e_edges = edge_index.shape[1]
    dpad = ((d_node + 127) // 128) * 128

    xp = jnp.pad(x, ((0, 0), (0, dpad - d_node)))
    wn = jnp.pad(W_node, ((0, dpad - d_node), (0, 0)))
    a_tab, b_tab = _tc_stage(
        xp, wn, b_node.reshape(1, ch), Wa1[:ch], Wa1[ch:], ba1.reshape(1, ch)
    )

    params = jnp.concatenate(
        [Wa2.reshape(-1), ba2.reshape(-1), jnp.zeros((LANES - 1,), jnp.float32)]
    )
    sc_stage = _make_sc_stage(n, ch, e_edges)
    out = sc_stage(a_tab, b_tab, edge_index[0], edge_index[1], params)
    return out.reshape(e_edges, 1)
